# baseline (speedup 1.0000x reference)
import functools

import jax
import jax.numpy as jnp
from jax import lax
from jax.experimental import pallas as pl
from jax.experimental.pallas import tpu as pltpu
from jax.experimental.pallas import tpu_sc as plsc

_BINS = 64
_B = 16
_N = 512 * 512
_NW = 32
_EPW = _B * _N // _NW
_RB = 32
_CB = _RB * 128
_NCHUNK = _EPW // _CB
_LANES = 16
_SLOTS = _BINS + 1
_LN2 = 0.6931471805599453


def _div(a, b):
    bv = jnp.ones((16,), jnp.float32) * b
    r = jnp.ones((16,), jnp.float32) / bv
    r = r * (jnp.float32(2.0) - bv * r)
    r = r * (jnp.float32(2.0) - bv * r)
    return a * r


def _ln(x):
    bits = plsc.bitcast(x, jnp.int32)
    e = ((bits >> 23) & 0xFF) - 127
    m = plsc.bitcast((bits & 0x7FFFFF) | 0x3F800000, jnp.float32)
    s = _div(m - jnp.float32(1.0), m + jnp.float32(1.0))
    t = s * s
    p = jnp.float32(1.0 / 11.0)
    for c in (1.0 / 9.0, 1.0 / 7.0, 1.0 / 5.0, 1.0 / 3.0, 1.0):
        p = p * t + jnp.float32(c)
    return jnp.float32(2.0) * s * p + e.astype(jnp.float32) * jnp.float32(_LN2)


def _hist_body(x_hbm, y_hbm, out_hbm, xb0, xb1, yb0, yb1, hist, hist2,
               oth, mi8, mivec, shared_h, shared_mi, sx0, sx1, sy0, sy1):
    cid = lax.axis_index("c")
    sid = lax.axis_index("s")
    sample = cid * 8 + sid // 2
    half = sid % 2
    lane = lax.iota(jnp.int32, 16)
    lane_base = lane * _SLOTS
    ones = jnp.ones((16,), jnp.float32)
    zeros = jnp.zeros((16,), jnp.float32)

    def _zero(i, _):
        hist[pl.ds(i * 16, 16)] = zeros
        return 0
    lax.fori_loop(0, _SLOTS, _zero, 0)

    xbufs = (xb0, xb1)
    ybufs = (yb0, yb1)
    sxs = (sx0, sx1)
    sys_ = (sy0, sy1)

    def _start(ck):
        slot = ck % 2
        rb = half * 256 + (ck // 4) * _RB
        cb = (ck % 4) * 128
        hx = pltpu.async_copy(
            x_hbm.at[sample, 0, pl.ds(rb, _RB), pl.ds(cb, 128)],
            xbufs[slot], sxs[slot])
        hy = pltpu.async_copy(
            y_hbm.at[sample, 0, pl.ds(rb, _RB), pl.ds(cb, 128)],
            ybufs[slot], sys_[slot])
        return hx, hy

    def _consume(ck, carry):
        slot = ck % 2
        xb = xbufs[slot]
        yb = ybufs[slot]

        def _inner(i, prev):
            pairs = []
            for k in range(8):
                xv = xb[i, pl.ds(k * 16, 16)]
                yv = yb[i, pl.ds(k * 16, 16)]
                v = xv * jnp.float32(_BINS) + yv
                iv = v.astype(jnp.int32)
                m = v <= jnp.float32(_BINS)
                pairs.append((iv + lane_base, m))
            for flat, m in zip(prev[0], prev[1]):
                plsc.addupdate_scatter(hist, [flat], ones, mask=m)
            return (tuple(p[0] for p in pairs), tuple(p[1] for p in pairs))
        return lax.fori_loop(0, _RB, _inner, carry)

    carry = (tuple(lane_base for _ in range(8)),
             tuple(lane < 0 for _ in range(8)))
    pending = _start(0)
    for ck in range(_NCHUNK):
        nxt = _start(ck + 1) if ck + 1 < _NCHUNK else None
        pending[0].wait()
        pending[1].wait()
        carry = _consume(ck, carry)
        pending = nxt
    for flat, m in zip(carry[0], carry[1]):
        plsc.addupdate_scatter(hist, [flat], ones, mask=m)

    for g in range(4):
        acc = hist[pl.ds(g * 16, 16)]
        for l in range(1, 16):
            acc = acc + hist[pl.ds(l * _SLOTS + g * 16, 16)]
        if g == 3:
            e64 = plsc.load_gather(hist, [lane * _SLOTS + (_SLOTS - 1)])
            s64 = jnp.sum(e64)
            acc = acc + jnp.where(lane == 15, s64, jnp.float32(0.0))
        hist2[pl.ds(g * 16, 16)] = acc

    pltpu.sync_copy(hist2, shared_h.at[sid])
    plsc.subcore_barrier()

    @pl.when(half == 0)
    def _mi():
        pltpu.sync_copy(shared_h.at[sid + 1], oth)
        hs = [hist2[pl.ds(g * 16, 16)] + oth[pl.ds(g * 16, 16)]
              for g in range(4)]
        tot = hs[0] + hs[1] + hs[2] + hs[3]
        tot = jnp.sum(tot)
        ps = [_div(h, tot) + jnp.float32(1e-8) for h in hs]
        s = jnp.sum(ps[0] + ps[1] + ps[2] + ps[3])
        den = s * s
        mi = jnp.float32(0.0)
        for p in ps:
            mi = mi + jnp.sum(p * _ln(_div(p, den)))
        mivec[...] = jnp.where(lane == 0, mi, jnp.float32(0.0))
        pltpu.sync_copy(mivec, shared_mi.at[sid // 2])

    plsc.subcore_barrier()

    @pl.when(sid == 0)
    def _reduce():
        pltpu.sync_copy(shared_mi, mi8)
        acc = mi8[0]
        for r2 in range(1, 8):
            acc = acc + mi8[r2]
        mivec[...] = acc * jnp.float32(-1.0 / _B)
        pltpu.sync_copy(mivec, out_hbm.at[cid])


_hist_sc = functools.partial(
    pl.kernel,
    out_type=jax.ShapeDtypeStruct((2, 16), jnp.float32),
    mesh=plsc.VectorSubcoreMesh(core_axis_name="c", subcore_axis_name="s"),
    compiler_params=pltpu.CompilerParams(
        needs_layout_passes=False, use_tc_tiling_on_sc=True),
    scratch_types=[
        pltpu.VMEM((_RB, 128), jnp.float32),
        pltpu.VMEM((_RB, 128), jnp.float32),
        pltpu.VMEM((_RB, 128), jnp.float32),
        pltpu.VMEM((_RB, 128), jnp.float32),
        pltpu.VMEM((_SLOTS * _LANES,), jnp.float32),
        pltpu.VMEM((_BINS,), jnp.float32),
        pltpu.VMEM((_BINS,), jnp.float32),
        pltpu.VMEM((8, 16), jnp.float32),
        pltpu.VMEM((16,), jnp.float32),
        pltpu.VMEM_SHARED((16, _BINS), jnp.float32),
        pltpu.VMEM_SHARED((8, 16), jnp.float32),
        pltpu.SemaphoreType.DMA,
        pltpu.SemaphoreType.DMA,
        pltpu.SemaphoreType.DMA,
        pltpu.SemaphoreType.DMA,
    ],
)(_hist_body)


def kernel(x, y):
    part = _hist_sc(x, y)
    return part[0, 0] + part[1, 0]


# speedup vs baseline: 1.1467x; 1.1467x over previous
import functools

import jax
import jax.numpy as jnp
from jax import lax
from jax.experimental import pallas as pl
from jax.experimental.pallas import tpu as pltpu
from jax.experimental.pallas import tpu_sc as plsc

_BINS = 64
_B = 16
_N = 512 * 512
_NW = 32
_EPW = _B * _N // _NW
_RB = 64
_CB = _RB * 128
_NCHUNK = _EPW // _CB
_LANES = 16
_SLOTS = _BINS + 1


def _hist_body(x_hbm, y_hbm, out_hbm, xb0, xb1, yb0, yb1, hist, hist2,
               sx0, sx1, sy0, sy1):
    cid = lax.axis_index("c")
    sid = lax.axis_index("s")
    wid = sid * 2 + cid
    sample = wid // 2
    half = wid % 2
    lane = lax.iota(jnp.int32, 16)
    lane_base = lane * _SLOTS
    ones = jnp.ones((16,), jnp.float32)
    zeros = jnp.zeros((16,), jnp.float32)

    def _zero(i, _):
        hist[pl.ds(i * 16, 16)] = zeros
        return 0
    lax.fori_loop(0, _SLOTS, _zero, 0)

    xbufs = (xb0, xb1)
    ybufs = (yb0, yb1)
    sxs = (sx0, sx1)
    sys_ = (sy0, sy1)

    def _start(ck):
        slot = ck % 2
        rb = half * 256 + (ck // 4) * _RB
        cb = (ck % 4) * 128
        hx = pltpu.async_copy(
            x_hbm.at[sample, 0, pl.ds(rb, _RB), pl.ds(cb, 128)],
            xbufs[slot], sxs[slot])
        hy = pltpu.async_copy(
            y_hbm.at[sample, 0, pl.ds(rb, _RB), pl.ds(cb, 128)],
            ybufs[slot], sys_[slot])
        return hx, hy

    def _consume(ck, carry):
        slot = ck % 2
        xb = xbufs[slot]
        yb = ybufs[slot]

        def _inner(i, prev):
            pairs = []
            for k in range(8):
                xv = xb[i, pl.ds(k * 16, 16)]
                yv = yb[i, pl.ds(k * 16, 16)]
                v = xv * jnp.float32(_BINS) + yv
                iv = v.astype(jnp.int32)
                m = v <= jnp.float32(_BINS)
                pairs.append((iv + lane_base, m))
            for flat, m in zip(prev[0], prev[1]):
                plsc.addupdate_scatter(hist, [flat], ones, mask=m)
            return (tuple(p[0] for p in pairs), tuple(p[1] for p in pairs))
        return lax.fori_loop(0, _RB, _inner, carry)

    carry = (tuple(lane_base for _ in range(8)),
             tuple(lane < 0 for _ in range(8)))
    pending = _start(0)
    for ck in range(_NCHUNK):
        nxt = _start(ck + 1) if ck + 1 < _NCHUNK else None
        pending[0].wait()
        pending[1].wait()
        carry = _consume(ck, carry)
        pending = nxt
    for flat, m in zip(carry[0], carry[1]):
        plsc.addupdate_scatter(hist, [flat], ones, mask=m)

    for g in range(4):
        acc = hist[pl.ds(g * 16, 16)]
        for l in range(1, 16):
            acc = acc + hist[pl.ds(l * _SLOTS + g * 16, 16)]
        if g == 3:
            e64 = plsc.load_gather(hist, [lane * _SLOTS + (_SLOTS - 1)])
            s64 = jnp.sum(e64)
            acc = acc + jnp.where(lane == 15, s64, jnp.float32(0.0))
        hist2[pl.ds(g * 16, 16)] = acc

    r = half * 16 + sample
    pltpu.sync_copy(hist2, out_hbm.at[r])


_hist_sc = functools.partial(
    pl.kernel,
    out_type=jax.ShapeDtypeStruct((_NW, _BINS), jnp.float32),
    mesh=plsc.VectorSubcoreMesh(core_axis_name="c", subcore_axis_name="s"),
    compiler_params=pltpu.CompilerParams(
        needs_layout_passes=False, use_tc_tiling_on_sc=True),
    scratch_types=[
        pltpu.VMEM((_RB, 128), jnp.float32),
        pltpu.VMEM((_RB, 128), jnp.float32),
        pltpu.VMEM((_RB, 128), jnp.float32),
        pltpu.VMEM((_RB, 128), jnp.float32),
        pltpu.VMEM((_SLOTS * _LANES,), jnp.float32),
        pltpu.VMEM((_BINS,), jnp.float32),
        pltpu.SemaphoreType.DMA,
        pltpu.SemaphoreType.DMA,
        pltpu.SemaphoreType.DMA,
        pltpu.SemaphoreType.DMA,
    ],
)(_hist_body)


def _mi_body(h_ref, o_ref):
    hcat = h_ref[...]
    h = hcat[0:16, :] + hcat[16:32, :]
    tot = jnp.sum(h, axis=1, keepdims=True)
    p = h / tot + jnp.float32(1e-8)
    s = jnp.sum(p, axis=1, keepdims=True)
    mi = p * jnp.log(p / (s * s))
    per_sample = jnp.sum(mi, axis=1, keepdims=True)
    total = jnp.sum(per_sample, axis=0, keepdims=True)
    o_ref[...] = -total / jnp.float32(_B)


_mi_tc = pl.pallas_call(
    _mi_body,
    out_shape=jax.ShapeDtypeStruct((1, 1), jnp.float32),
)


def kernel(x, y):
    part = _hist_sc(x, y)
    return _mi_tc(part)[0, 0]


# speedup vs baseline: 1.1468x; 1.0001x over previous
import functools

import jax
import jax.numpy as jnp
from jax import lax
from jax.experimental import pallas as pl
from jax.experimental.pallas import tpu as pltpu
from jax.experimental.pallas import tpu_sc as plsc

_BINS = 64
_B = 16
_N = 512 * 512
_NW = 32
_EPW = _B * _N // _NW
_RB = 128
_CB = _RB * 128
_NCHUNK = _EPW // _CB
_LANES = 16
_SLOTS = _BINS + 1


def _hist_body(x_hbm, y_hbm, out_hbm, xb0, xb1, yb0, yb1, hist, hist2,
               sx0, sx1, sy0, sy1):
    cid = lax.axis_index("c")
    sid = lax.axis_index("s")
    wid = sid * 2 + cid
    sample = wid // 2
    half = wid % 2
    lane = lax.iota(jnp.int32, 16)
    lane_base = lane * _SLOTS
    ones = jnp.ones((16,), jnp.float32)
    zeros = jnp.zeros((16,), jnp.float32)

    def _zero(i, _):
        hist[pl.ds(i * 16, 16)] = zeros
        return 0
    lax.fori_loop(0, _SLOTS, _zero, 0)

    xbufs = (xb0, xb1)
    ybufs = (yb0, yb1)
    sxs = (sx0, sx1)
    sys_ = (sy0, sy1)

    def _start(ck):
        slot = ck % 2
        rb = half * 256 + (ck // 4) * _RB
        cb = (ck % 4) * 128
        hx = pltpu.async_copy(
            x_hbm.at[sample, 0, pl.ds(rb, _RB), pl.ds(cb, 128)],
            xbufs[slot], sxs[slot])
        hy = pltpu.async_copy(
            y_hbm.at[sample, 0, pl.ds(rb, _RB), pl.ds(cb, 128)],
            ybufs[slot], sys_[slot])
        return hx, hy

    def _consume(ck, carry):
        slot = ck % 2
        xb = xbufs[slot]
        yb = ybufs[slot]

        def _inner(i, prev):
            pairs = []
            for k in range(8):
                xv = xb[i, pl.ds(k * 16, 16)]
                yv = yb[i, pl.ds(k * 16, 16)]
                v = xv * jnp.float32(_BINS) + yv
                iv = v.astype(jnp.int32)
                m = v <= jnp.float32(_BINS)
                pairs.append((iv + lane_base, m))
            for flat, m in zip(prev[0], prev[1]):
                plsc.addupdate_scatter(hist, [flat], ones, mask=m)
            return (tuple(p[0] for p in pairs), tuple(p[1] for p in pairs))
        return lax.fori_loop(0, _RB, _inner, carry)

    carry = (tuple(lane_base for _ in range(8)),
             tuple(lane < 0 for _ in range(8)))
    pending = _start(0)
    for ck in range(_NCHUNK):
        nxt = _start(ck + 1) if ck + 1 < _NCHUNK else None
        pending[0].wait()
        pending[1].wait()
        carry = _consume(ck, carry)
        pending = nxt
    for flat, m in zip(carry[0], carry[1]):
        plsc.addupdate_scatter(hist, [flat], ones, mask=m)

    for g in range(4):
        acc = hist[pl.ds(g * 16, 16)]
        for l in range(1, 16):
            acc = acc + hist[pl.ds(l * _SLOTS + g * 16, 16)]
        if g == 3:
            e64 = plsc.load_gather(hist, [lane * _SLOTS + (_SLOTS - 1)])
            s64 = jnp.sum(e64)
            acc = acc + jnp.where(lane == 15, s64, jnp.float32(0.0))
        hist2[pl.ds(g * 16, 16)] = acc

    r = half * 16 + sample
    pltpu.sync_copy(hist2, out_hbm.at[r])


_hist_sc = functools.partial(
    pl.kernel,
    out_type=jax.ShapeDtypeStruct((_NW, _BINS), jnp.float32),
    mesh=plsc.VectorSubcoreMesh(core_axis_name="c", subcore_axis_name="s"),
    compiler_params=pltpu.CompilerParams(
        needs_layout_passes=False, use_tc_tiling_on_sc=True),
    scratch_types=[
        pltpu.VMEM((_RB, 128), jnp.float32),
        pltpu.VMEM((_RB, 128), jnp.float32),
        pltpu.VMEM((_RB, 128), jnp.float32),
        pltpu.VMEM((_RB, 128), jnp.float32),
        pltpu.VMEM((_SLOTS * _LANES,), jnp.float32),
        pltpu.VMEM((_BINS,), jnp.float32),
        pltpu.SemaphoreType.DMA,
        pltpu.SemaphoreType.DMA,
        pltpu.SemaphoreType.DMA,
        pltpu.SemaphoreType.DMA,
    ],
)(_hist_body)


def _mi_body(h_ref, o_ref):
    hcat = h_ref[...]
    h = hcat[0:16, :] + hcat[16:32, :]
    tot = jnp.sum(h, axis=1, keepdims=True)
    p = h / tot + jnp.float32(1e-8)
    s = jnp.sum(p, axis=1, keepdims=True)
    mi = p * jnp.log(p / (s * s))
    per_sample = jnp.sum(mi, axis=1, keepdims=True)
    total = jnp.sum(per_sample, axis=0, keepdims=True)
    o_ref[...] = -total / jnp.float32(_B)


_mi_tc = pl.pallas_call(
    _mi_body,
    out_shape=jax.ShapeDtypeStruct((1, 1), jnp.float32),
)


def kernel(x, y):
    part = _hist_sc(x, y)
    return _mi_tc(part)[0, 0]
